# Initial kernel scaffold; baseline (speedup 1.0000x reference)
import jax
import jax.numpy as jnp
from jax.experimental import pallas as pl


def kernel(x, W_sw, b_sw, W_in, b_in, W_out, b_out):
    raise NotImplementedError("write your pallas kernel here")


# speedup vs baseline: 2.6991x; 2.6991x over previous
import functools

import jax
import jax.numpy as jnp
from jax import lax
from jax.experimental import pallas as pl
from jax.experimental.pallas import tpu as pltpu
from jax.experimental.pallas import tpu_sc as plsc

N = 4096
D = 1024
F = 4096
E = 8
TR = 512
NTR = N // TR
TM = 512
P = N + E * TM
NT = P // TM


def _router_body(x_ref, w_ref, b_ref, routes_ref, ranks_ref, tcnt_ref, ps_ref):
    i = pl.program_id(0)
    x = x_ref[...]
    logits = jnp.dot(x, w_ref[...], preferred_element_type=jnp.float32)
    logits = logits + b_ref[...]
    m = jnp.max(logits, axis=1, keepdims=True)
    p = jnp.exp(logits - m)
    s = jnp.sum(p, axis=1, keepdims=True)
    prob = p / s

    eidx = lax.broadcasted_iota(jnp.int32, (TR, E), 1)
    route = jnp.min(jnp.where(logits == m, eidx, E), axis=1)
    oh = (eidx == route[:, None]).astype(jnp.float32)

    r_i = lax.broadcasted_iota(jnp.int32, (TR, TR), 0)
    c_i = lax.broadcasted_iota(jnp.int32, (TR, TR), 1)
    ltri = (r_i > c_i).astype(jnp.float32)
    prior = jnp.dot(ltri, oh, preferred_element_type=jnp.float32)
    rank = jnp.sum(prior * oh, axis=1)

    routes_ref[0, 0, :] = route
    ranks_ref[0, 0, :] = rank.astype(jnp.int32)
    tcnt_ref[0, 0, :] = jnp.sum(oh, axis=0).astype(jnp.int32)

    @pl.when(i == 0)
    def _():
        ps_ref[...] = jnp.zeros_like(ps_ref)

    ps_ref[...] += jnp.sum(prob, axis=0, keepdims=True)


def _run_router(xf, W_sw, b_sw):
    return pl.pallas_call(
        _router_body,
        grid=(NTR,),
        in_specs=[
            pl.BlockSpec((TR, D), lambda i: (i, 0)),
            pl.BlockSpec((D, E), lambda i: (0, 0)),
            pl.BlockSpec((1, E), lambda i: (0, 0)),
        ],
        out_specs=[
            pl.BlockSpec((1, 1, TR), lambda i: (i, 0, 0)),
            pl.BlockSpec((1, 1, TR), lambda i: (i, 0, 0)),
            pl.BlockSpec((1, 1, E), lambda i: (i, 0, 0)),
            pl.BlockSpec((1, E), lambda i: (0, 0)),
        ],
        out_shape=[
            jax.ShapeDtypeStruct((NTR, 1, TR), jnp.int32),
            jax.ShapeDtypeStruct((NTR, 1, TR), jnp.int32),
            jax.ShapeDtypeStruct((NTR, 1, E), jnp.int32),
            jax.ShapeDtypeStruct((1, E), jnp.float32),
        ],
    )(xf, W_sw, b_sw.reshape(1, E))


def _plan_body(routes_ref, ranks_ref, tcnt_ref,
               slot_ref, te_ref, na_ref, cnt_ref):
    tc = tcnt_ref[...].reshape(NTR, E).astype(jnp.float32)
    r_i = lax.broadcasted_iota(jnp.int32, (NTR, NTR), 0)
    c_i = lax.broadcasted_iota(jnp.int32, (NTR, NTR), 1)
    ltri = (r_i > c_i).astype(jnp.float32)
    tile_base = jnp.dot(ltri, tc, preferred_element_type=jnp.float32)
    tile_base_i = tile_base.astype(jnp.int32)

    counts = jnp.sum(tc, axis=0)
    counts_i = counts.astype(jnp.int32)
    pc = ((counts_i + (TM - 1)) // TM) * TM
    re_ = lax.broadcasted_iota(jnp.int32, (E, E), 0)
    ce_ = lax.broadcasted_iota(jnp.int32, (E, E), 1)
    ltri_e = (re_ > ce_).astype(jnp.float32)
    pb = jnp.dot(ltri_e, pc.astype(jnp.float32)[:, None],
                 preferred_element_type=jnp.float32)[:, 0]
    pb_i = pb.astype(jnp.int32)

    routes = routes_ref[...].reshape(NTR, TR)
    ranks = ranks_ref[...].reshape(NTR, TR)
    eidx = lax.broadcasted_iota(jnp.int32, (TR, E), 1)
    for k in range(NTR):
        r = routes[k]
        oh = (eidx == r[:, None])
        pb_t = jnp.sum(jnp.where(oh, pb_i[None, :], 0), axis=1)
        tb_t = jnp.sum(jnp.where(oh, tile_base_i[k][None, :], 0), axis=1)
        slot_ref[k, 0, :] = pb_t + tb_t + ranks[k]

    ends_t = (pb_i + pc) // TM
    jt = lax.broadcasted_iota(jnp.int32, (NT, E), 0)
    te = jnp.sum((ends_t[None, :] <= jt).astype(jnp.int32), axis=1)
    te_ref[0, :] = jnp.minimum(te, E - 1)
    na_ref[0, :] = jnp.full((E,), jnp.sum(pc) // TM, jnp.int32)
    cnt_ref[0, :] = counts


def _run_plan(routes, ranks, tcnt):
    return pl.pallas_call(
        _plan_body,
        grid=(1,),
        in_specs=[
            pl.BlockSpec((NTR, 1, TR), lambda i: (0, 0, 0)),
            pl.BlockSpec((NTR, 1, TR), lambda i: (0, 0, 0)),
            pl.BlockSpec((NTR, 1, E), lambda i: (0, 0, 0)),
        ],
        out_specs=[
            pl.BlockSpec((NTR, 1, TR), lambda i: (0, 0, 0)),
            pl.BlockSpec((1, NT), lambda i: (0, 0)),
            pl.BlockSpec((1, E), lambda i: (0, 0)),
            pl.BlockSpec((1, E), lambda i: (0, 0)),
        ],
        out_shape=[
            jax.ShapeDtypeStruct((NTR, 1, TR), jnp.int32),
            jax.ShapeDtypeStruct((1, NT), jnp.int32),
            jax.ShapeDtypeStruct((1, E), jnp.int32),
            jax.ShapeDtypeStruct((1, E), jnp.float32),
        ],
    )(routes, ranks, tcnt)


_NC, _NS = 2, 16
NW = _NC * _NS


def _scatter_rows(xf, slot_flat, chunk=64):
    per_w = N // NW
    n_chunks = per_w // chunk
    mesh = plsc.VectorSubcoreMesh(core_axis_name="c", subcore_axis_name="s")

    @functools.partial(
        pl.kernel, mesh=mesh,
        out_type=jax.ShapeDtypeStruct((P, D), jnp.float32),
        scratch_types=[
            pltpu.VMEM((n_chunks, chunk), jnp.int32),
            pltpu.VMEM((chunk, D), jnp.float32),
            pltpu.SemaphoreType.DMA,
        ],
    )
    def k(xf_hbm, slot_hbm, out_hbm, idx_v, rows_v, sem):
        wid = lax.axis_index("s") * _NC + lax.axis_index("c")
        base = wid * per_w
        for c in range(n_chunks):
            pltpu.sync_copy(slot_hbm.at[pl.ds(base + c * chunk, chunk)],
                            idx_v.at[c])
            pltpu.sync_copy(xf_hbm.at[pl.ds(base + c * chunk, chunk)], rows_v)
            pltpu.async_copy(rows_v, out_hbm.at[idx_v.at[c]], sem).wait()

    return k(xf, slot_flat)


def _gather_rows(table, idx, n_rows, chunk=64):
    per_w = n_rows // NW
    n_chunks = per_w // chunk
    mesh = plsc.VectorSubcoreMesh(core_axis_name="c", subcore_axis_name="s")

    @functools.partial(
        pl.kernel, mesh=mesh,
        out_type=jax.ShapeDtypeStruct((n_rows, D), jnp.float32),
        scratch_types=[
            pltpu.VMEM((n_chunks, chunk), jnp.int32),
            pltpu.VMEM((chunk, D), jnp.float32),
            pltpu.SemaphoreType.DMA,
        ],
    )
    def k(table_hbm, idx_hbm, out_hbm, idx_v, rows_v, sem):
        wid = lax.axis_index("s") * _NC + lax.axis_index("c")
        base = wid * per_w
        for c in range(n_chunks):
            pltpu.sync_copy(idx_hbm.at[pl.ds(base + c * chunk, chunk)],
                            idx_v.at[c])
            pltpu.async_copy(table_hbm.at[idx_v.at[c]], rows_v, sem).wait()
            pltpu.sync_copy(rows_v, out_hbm.at[pl.ds(base + c * chunk, chunk)])

    return k(table, idx)


def _ffn_body(te_ref, na_ref, x_ref, wi_ref, bi_ref, wo_ref, bo_ref, out_ref):
    @pl.when(pl.program_id(0) < na_ref[0])
    def _():
        xb = x_ref[...].astype(jnp.bfloat16)
        h = jnp.dot(xb, wi_ref[0], preferred_element_type=jnp.float32)
        h = jnp.maximum(h + bi_ref[0], 0.0).astype(jnp.bfloat16)
        out = jnp.dot(h, wo_ref[0], preferred_element_type=jnp.float32)
        out_ref[...] = out + bo_ref[0]


def _run_ffn(x_sorted, W_in, b_in, W_out, b_out, tile_expert, n_active):
    grid_spec = pltpu.PrefetchScalarGridSpec(
        num_scalar_prefetch=2,
        grid=(NT,),
        in_specs=[
            pl.BlockSpec((TM, D), lambda i, te, na: (i, 0)),
            pl.BlockSpec((1, D, F), lambda i, te, na: (te[i], 0, 0)),
            pl.BlockSpec((1, 1, F), lambda i, te, na: (te[i], 0, 0)),
            pl.BlockSpec((1, F, D), lambda i, te, na: (te[i], 0, 0)),
            pl.BlockSpec((1, 1, D), lambda i, te, na: (te[i], 0, 0)),
        ],
        out_specs=pl.BlockSpec((TM, D), lambda i, te, na: (i, 0)),
    )
    return pl.pallas_call(
        _ffn_body,
        grid_spec=grid_spec,
        out_shape=jax.ShapeDtypeStruct((P, D), jnp.float32),
    )(tile_expert, n_active, x_sorted, W_in, b_in, W_out, b_out)


def kernel(x, W_sw, b_sw, W_in, b_in, W_out, b_out):
    b, s, d = x.shape
    xf = x.reshape(-1, d)

    routes, ranks, tcnt, prob_sum = _run_router(xf, W_sw, b_sw)
    slot, tile_expert, n_active, counts = _run_plan(routes, ranks, tcnt)

    slot_flat = slot.reshape(N)
    x_sorted = _scatter_rows(xf, slot_flat)

    out_sorted = _run_ffn(
        x_sorted,
        W_in.astype(jnp.bfloat16), b_in.reshape(E, 1, F),
        W_out.astype(jnp.bfloat16), b_out.reshape(E, 1, D),
        tile_expert.reshape(NT), n_active.reshape(E)[:1],
    )

    final = _gather_rows(out_sorted, slot_flat, N)
    return (final.reshape(b, s, d), counts.reshape(E),
            prob_sum.reshape(E), 0)


# speedup vs baseline: 2.7092x; 1.0037x over previous
import functools

import jax
import jax.numpy as jnp
from jax import lax
from jax.experimental import pallas as pl
from jax.experimental.pallas import tpu as pltpu
from jax.experimental.pallas import tpu_sc as plsc

N = 4096
D = 1024
F = 4096
E = 8
TR = 512
NTR = N // TR
TM = 512
P = N + E * TM
NT = P // TM


def _router_body(x_ref, w_ref, b_ref,
                 ps_ref, slot_ref, te_ref, na_ref, cnt_ref,
                 routes_s, ranks_s, tcnt_s):
    i = pl.program_id(0)
    x = x_ref[...]
    logits = jnp.dot(x, w_ref[...], preferred_element_type=jnp.float32)
    logits = logits + b_ref[...]
    m = jnp.max(logits, axis=1, keepdims=True)
    p = jnp.exp(logits - m)
    s = jnp.sum(p, axis=1, keepdims=True)
    prob = p / s

    eidx = lax.broadcasted_iota(jnp.int32, (TR, E), 1)
    route = jnp.min(jnp.where(logits == m, eidx, E), axis=1)
    oh = (eidx == route[:, None]).astype(jnp.float32)

    r_i = lax.broadcasted_iota(jnp.int32, (TR, TR), 0)
    c_i = lax.broadcasted_iota(jnp.int32, (TR, TR), 1)
    ltri = (r_i > c_i).astype(jnp.float32)
    prior = jnp.dot(ltri, oh, preferred_element_type=jnp.float32)
    rank = jnp.sum(prior * oh, axis=1)

    routes_s[i, 0, :] = route
    ranks_s[i, 0, :] = rank.astype(jnp.int32)
    tcnt_s[i, 0, :] = jnp.sum(oh, axis=0).astype(jnp.int32)

    @pl.when(i == 0)
    def _():
        ps_ref[...] = jnp.zeros_like(ps_ref)

    ps_ref[...] += jnp.sum(prob, axis=0, keepdims=True)

    @pl.when(i == NTR - 1)
    def _():
        tc = tcnt_s[...].reshape(NTR, E).astype(jnp.float32)
        r8 = lax.broadcasted_iota(jnp.int32, (NTR, NTR), 0)
        c8 = lax.broadcasted_iota(jnp.int32, (NTR, NTR), 1)
        ltri8 = (r8 > c8).astype(jnp.float32)
        tile_base_i = jnp.dot(ltri8, tc,
                              preferred_element_type=jnp.float32
                              ).astype(jnp.int32)

        counts = jnp.sum(tc, axis=0)
        counts_i = counts.astype(jnp.int32)
        pc = ((counts_i + (TM - 1)) // TM) * TM
        re_ = lax.broadcasted_iota(jnp.int32, (E, E), 0)
        ce_ = lax.broadcasted_iota(jnp.int32, (E, E), 1)
        ltri_e = (re_ > ce_).astype(jnp.float32)
        pb_i = jnp.dot(ltri_e, pc.astype(jnp.float32)[:, None],
                       preferred_element_type=jnp.float32
                       )[:, 0].astype(jnp.int32)

        routes = routes_s[...].reshape(NTR, TR)
        ranks = ranks_s[...].reshape(NTR, TR)
        for k in range(NTR):
            ohk = (eidx == routes[k][:, None])
            pb_t = jnp.sum(jnp.where(ohk, pb_i[None, :], 0), axis=1)
            tb_t = jnp.sum(jnp.where(ohk, tile_base_i[k][None, :], 0), axis=1)
            slot_ref[k, 0, :] = pb_t + tb_t + ranks[k]

        ends_t = (pb_i + pc) // TM
        jt = lax.broadcasted_iota(jnp.int32, (NT, E), 0)
        te = jnp.sum((ends_t[None, :] <= jt).astype(jnp.int32), axis=1)
        te_ref[0, :] = jnp.minimum(te, E - 1)
        na_ref[0, :] = jnp.full((E,), jnp.sum(pc) // TM, jnp.int32)
        cnt_ref[0, :] = counts


def _run_router(xf, W_sw, b_sw):
    return pl.pallas_call(
        _router_body,
        grid=(NTR,),
        in_specs=[
            pl.BlockSpec((TR, D), lambda i: (i, 0)),
            pl.BlockSpec((D, E), lambda i: (0, 0)),
            pl.BlockSpec((1, E), lambda i: (0, 0)),
        ],
        out_specs=[
            pl.BlockSpec((1, E), lambda i: (0, 0)),
            pl.BlockSpec((NTR, 1, TR), lambda i: (0, 0, 0)),
            pl.BlockSpec((1, NT), lambda i: (0, 0)),
            pl.BlockSpec((1, E), lambda i: (0, 0)),
            pl.BlockSpec((1, E), lambda i: (0, 0)),
        ],
        out_shape=[
            jax.ShapeDtypeStruct((1, E), jnp.float32),
            jax.ShapeDtypeStruct((NTR, 1, TR), jnp.int32),
            jax.ShapeDtypeStruct((1, NT), jnp.int32),
            jax.ShapeDtypeStruct((1, E), jnp.int32),
            jax.ShapeDtypeStruct((1, E), jnp.float32),
        ],
        scratch_shapes=[
            pltpu.VMEM((NTR, 1, TR), jnp.int32),
            pltpu.VMEM((NTR, 1, TR), jnp.int32),
            pltpu.VMEM((NTR, 1, E), jnp.int32),
        ],
    )(xf, W_sw, b_sw.reshape(1, E))


_NC, _NS = 2, 16
NW = _NC * _NS


def _scatter_rows(xf, slot_flat, chunk=64):
    per_w = N // NW
    n_chunks = per_w // chunk
    mesh = plsc.VectorSubcoreMesh(core_axis_name="c", subcore_axis_name="s")

    @functools.partial(
        pl.kernel, mesh=mesh,
        out_type=jax.ShapeDtypeStruct((P, D), jnp.float32),
        scratch_types=[
            pltpu.VMEM((n_chunks, chunk), jnp.int32),
            pltpu.VMEM((chunk, D), jnp.float32),
            pltpu.SemaphoreType.DMA,
        ],
    )
    def k(xf_hbm, slot_hbm, out_hbm, idx_v, rows_v, sem):
        wid = lax.axis_index("s") * _NC + lax.axis_index("c")
        base = wid * per_w
        for c in range(n_chunks):
            pltpu.sync_copy(slot_hbm.at[pl.ds(base + c * chunk, chunk)],
                            idx_v.at[c])
            pltpu.sync_copy(xf_hbm.at[pl.ds(base + c * chunk, chunk)], rows_v)
            pltpu.async_copy(rows_v, out_hbm.at[idx_v.at[c]], sem).wait()

    return k(xf, slot_flat)


def _gather_rows(table, idx, n_rows, chunk=64):
    per_w = n_rows // NW
    n_chunks = per_w // chunk
    mesh = plsc.VectorSubcoreMesh(core_axis_name="c", subcore_axis_name="s")

    @functools.partial(
        pl.kernel, mesh=mesh,
        out_type=jax.ShapeDtypeStruct((n_rows, D), jnp.float32),
        scratch_types=[
            pltpu.VMEM((n_chunks, chunk), jnp.int32),
            pltpu.VMEM((chunk, D), jnp.float32),
            pltpu.SemaphoreType.DMA,
        ],
    )
    def k(table_hbm, idx_hbm, out_hbm, idx_v, rows_v, sem):
        wid = lax.axis_index("s") * _NC + lax.axis_index("c")
        base = wid * per_w
        for c in range(n_chunks):
            pltpu.sync_copy(idx_hbm.at[pl.ds(base + c * chunk, chunk)],
                            idx_v.at[c])
            pltpu.async_copy(table_hbm.at[idx_v.at[c]], rows_v, sem).wait()
            pltpu.sync_copy(rows_v, out_hbm.at[pl.ds(base + c * chunk, chunk)])

    return k(table, idx)


def _ffn_body(te_ref, na_ref, x_ref, wi_ref, bi_ref, wo_ref, bo_ref, out_ref):
    @pl.when(pl.program_id(0) < na_ref[0])
    def _():
        xb = x_ref[...].astype(jnp.bfloat16)
        h = jnp.dot(xb, wi_ref[0], preferred_element_type=jnp.float32)
        h = jnp.maximum(h + bi_ref[0], 0.0).astype(jnp.bfloat16)
        out = jnp.dot(h, wo_ref[0], preferred_element_type=jnp.float32)
        out_ref[...] = out + bo_ref[0]


def _run_ffn(x_sorted, W_in, b_in, W_out, b_out, tile_expert, n_active):
    grid_spec = pltpu.PrefetchScalarGridSpec(
        num_scalar_prefetch=2,
        grid=(NT,),
        in_specs=[
            pl.BlockSpec((TM, D), lambda i, te, na: (i, 0)),
            pl.BlockSpec((1, D, F), lambda i, te, na: (te[i], 0, 0)),
            pl.BlockSpec((1, 1, F), lambda i, te, na: (te[i], 0, 0)),
            pl.BlockSpec((1, F, D), lambda i, te, na: (te[i], 0, 0)),
            pl.BlockSpec((1, 1, D), lambda i, te, na: (te[i], 0, 0)),
        ],
        out_specs=pl.BlockSpec((TM, D), lambda i, te, na: (i, 0)),
    )
    return pl.pallas_call(
        _ffn_body,
        grid_spec=grid_spec,
        out_shape=jax.ShapeDtypeStruct((P, D), jnp.float32),
    )(tile_expert, n_active, x_sorted, W_in, b_in, W_out, b_out)


def kernel(x, W_sw, b_sw, W_in, b_in, W_out, b_out):
    b, s, d = x.shape
    xf = x.reshape(-1, d)

    prob_sum, slot, tile_expert, n_active, counts = _run_router(xf, W_sw, b_sw)

    slot_flat = slot.reshape(N)
    x_sorted = _scatter_rows(xf, slot_flat)

    out_sorted = _run_ffn(
        x_sorted,
        W_in.astype(jnp.bfloat16), b_in.reshape(E, 1, F),
        W_out.astype(jnp.bfloat16), b_out.reshape(E, 1, D),
        tile_expert.reshape(NT), n_active.reshape(E)[:1],
    )

    final = _gather_rows(out_sorted, slot_flat, N)
    return (final.reshape(b, s, d), counts.reshape(E),
            prob_sum.reshape(E), 0)
